# baseline (speedup 1.0000x reference)
import jax
import jax.numpy as jnp
from jax import lax
from jax.experimental import pallas as pl
from jax.experimental.pallas import tpu as pltpu
from jax.experimental.pallas import tpu_sc as plsc

N = 10000
E = 320000
D = 128

NC = 2
NS = 16
NW = NC * NS
LANES = 16

KB = 120
NB = 84
EPAD = NW * NB * KB
ZCH = KB
NZ = N // ZCH
NREM = N - NZ * ZCH


def _mm_body(x_ref, w_ref, o_ref):
    o_ref[...] = jnp.dot(x_ref[...], w_ref[...],
                         preferred_element_type=jnp.float32)


def _matmul(x, W):
    m_blk = 1000
    return pl.pallas_call(
        _mm_body,
        grid=(N // m_blk,),
        in_specs=[
            pl.BlockSpec((m_blk, D), lambda i: (i, 0)),
            pl.BlockSpec((D, D), lambda i: (0, 0)),
        ],
        out_specs=pl.BlockSpec((m_blk, D), lambda i: (i, 0)),
        out_shape=jax.ShapeDtypeStruct((N, D), jnp.float32),
    )(x, W)


def _sc_body(h_hbm, pk_hbm, out_hbm,
             acc, r0, r1, r2, st0, st1, st2,
             g0, g1, g2, s0, s1, s2, i0, i1, i2):
    cid = lax.axis_index("c")
    sid = lax.axis_index("s")
    gwid = cid * NS + sid
    rows = (r0, r1, r2)
    stg = (st0, st1, st2)
    gsem = (g0, g1, g2)
    ssem = (s0, s1, s2)
    isem = (i0, i1, i2)

    def _zero_row(r, _):
        for j in range(D // LANES):
            r0[r, pl.ds(j * LANES, LANES)] = jnp.zeros((LANES,), jnp.float32)
        return 0
    lax.fori_loop(0, KB, _zero_row, 0)

    def _zinit(i, _):
        blk = sid + i * NS

        @pl.when(blk < NZ)
        def _():
            pltpu.sync_copy(r0, acc.at[pl.ds(blk * ZCH, ZCH)])
        return 0
    lax.fori_loop(0, (NZ + NS - 1) // NS, _zinit, 0)

    @pl.when(sid == 0)
    def _():
        pltpu.sync_copy(r0.at[pl.ds(0, NREM)],
                        acc.at[pl.ds(NZ * ZCH, NREM)])
    plsc.subcore_barrier()

    def _stage_start(blk, b):
        pltpu.async_copy(pk_hbm.at[gwid, blk], stg[b], isem[b])

    def _stage_wait(blk, b):
        pltpu.make_async_copy(pk_hbm.at[gwid, blk], stg[b], isem[b]).wait()

    def _gather_start(blk, b):
        pltpu.async_copy(h_hbm.at[stg[b].at[0]], rows[b], gsem[b])

    def _gather_wait(blk, b):
        pltpu.make_async_copy(h_hbm.at[stg[b].at[0]], rows[b], gsem[b]).wait()

    def _scatter_start(blk, b):
        pltpu.async_copy(rows[b], acc.at[stg[b].at[1]], ssem[b], add=True)

    def _scatter_wait(blk, b):
        pltpu.make_async_copy(rows[b], acc.at[stg[b].at[1]], ssem[b]).wait()

    _stage_start(0, 0)
    _stage_start(1, 1)
    _stage_wait(0, 0)
    _gather_start(0, 0)

    def _trip(q, _):
        for b in range(3):
            blk = 3 * q + b
            _gather_wait(blk, b)

            bn = (b + 1) % 3
            if b < 2:
                _stage_wait(blk + 1, bn)
                _gather_start(blk + 1, bn)
            else:
                @pl.when(blk + 1 < NB)
                def _():
                    _stage_wait(blk + 1, bn)
                    _gather_start(blk + 1, bn)

            def _scale(c, _):
                w16 = lax.bitcast_convert_type(
                    stg[b][2, pl.ds(c * LANES, LANES)], jnp.float32)
                for lane in range(LANES):
                    wbc = w16.at[jnp.full((LANES,), lane, jnp.int32)].get(
                        mode="promise_in_bounds")
                    e = c * LANES + lane
                    for jj in range(D // LANES):
                        sl = pl.ds(jj * LANES, LANES)
                        rows[b][e, sl] = rows[b][e, sl] * wbc
                return 0
            lax.fori_loop(0, KB // LANES, _scale, 0)

            _scatter_start(blk, b)

            bp = (b + 2) % 3
            if b == 0:
                @pl.when(blk >= 1)
                def _():
                    _scatter_wait(blk - 1, bp)
            else:
                _scatter_wait(blk - 1, bp)

            if b == 0:
                _stage_start(blk + 2, bp)
            else:
                @pl.when(blk + 2 < NB)
                def _():
                    _stage_start(blk + 2, bp)
        return 0

    lax.fori_loop(0, NB // 3, _trip, 0)
    _scatter_wait(NB - 1, (NB - 1) % 3)
    plsc.subcore_barrier()

    def _wout(i, _):
        blk = sid + i * NS

        @pl.when(blk < NZ)
        def _():
            pltpu.sync_copy(acc.at[pl.ds(blk * ZCH, ZCH)],
                            out_hbm.at[cid, pl.ds(blk * ZCH, ZCH)])
        return 0
    lax.fori_loop(0, (NZ + NS - 1) // NS, _wout, 0)

    @pl.when(sid == 0)
    def _():
        pltpu.sync_copy(acc.at[pl.ds(NZ * ZCH, NREM)],
                        out_hbm.at[cid, pl.ds(NZ * ZCH, NREM)])


def _sc_aggregate(h, packed):
    mesh = plsc.VectorSubcoreMesh(core_axis_name="c", subcore_axis_name="s")
    f = pl.kernel(
        _sc_body,
        out_type=jax.ShapeDtypeStruct((NC, N, D), jnp.float32),
        mesh=mesh,
        scratch_types=(
            [pltpu.VMEM_SHARED((N, D), jnp.float32)]
            + [pltpu.VMEM((KB, D), jnp.float32) for _ in range(3)]
            + [pltpu.VMEM((3, KB), jnp.int32) for _ in range(3)]
            + [pltpu.SemaphoreType.DMA for _ in range(9)]
        ),
    )
    return f(h, packed)


def _comb_body(p_ref, b_ref, o_ref):
    o_ref[...] = p_ref[0] + p_ref[1] + b_ref[...]


def _combine(partials, b):
    m_blk = 1000
    return pl.pallas_call(
        _comb_body,
        grid=(N // m_blk,),
        in_specs=[
            pl.BlockSpec((NC, m_blk, D), lambda i: (0, i, 0)),
            pl.BlockSpec((1, D), lambda i: (0, 0)),
        ],
        out_specs=pl.BlockSpec((m_blk, D), lambda i: (i, 0)),
        out_shape=jax.ShapeDtypeStruct((N, D), jnp.float32),
    )(partials, b.reshape(1, D))


@jax.jit
def kernel(x, edge_index, edge_weight, W, b):
    h = _matmul(x, W)
    pad = EPAD - E
    src = jnp.concatenate([edge_index[0], jnp.zeros((pad,), jnp.int32)])
    dst = jnp.concatenate([edge_index[1], jnp.zeros((pad,), jnp.int32)])
    wbits = lax.bitcast_convert_type(
        jnp.concatenate([edge_weight, jnp.zeros((pad,), jnp.float32)]),
        jnp.int32)
    packed = jnp.stack(
        [src.reshape(NW, NB, KB),
         dst.reshape(NW, NB, KB),
         wbits.reshape(NW, NB, KB)], axis=2)

    partials = _sc_aggregate(h, packed)
    return _combine(partials, b)


# speedup vs baseline: 1.0391x; 1.0391x over previous
import jax
import jax.numpy as jnp
from jax import lax
from jax.experimental import pallas as pl
from jax.experimental.pallas import tpu as pltpu
from jax.experimental.pallas import tpu_sc as plsc

N = 10000
E = 320000
D = 128

NC = 2
NS = 16
NW = NC * NS
LANES = 16

KB = 112
NB = 90
EPAD = NW * NB * KB
ZCH = KB
NZ = N // ZCH
NREM = N - NZ * ZCH


def _mm_body(x_ref, w_ref, o_ref):
    o_ref[...] = jnp.dot(x_ref[...], w_ref[...],
                         preferred_element_type=jnp.float32)


def _matmul(x, W):
    m_blk = 1000
    return pl.pallas_call(
        _mm_body,
        grid=(N // m_blk,),
        in_specs=[
            pl.BlockSpec((m_blk, D), lambda i: (i, 0)),
            pl.BlockSpec((D, D), lambda i: (0, 0)),
        ],
        out_specs=pl.BlockSpec((m_blk, D), lambda i: (i, 0)),
        out_shape=jax.ShapeDtypeStruct((N, D), jnp.float32),
    )(x, W)


def _sc_body(h_hbm, src_hbm, dst_hbm, w_hbm, out_hbm,
             acc, r0, r1, r2, sv0, sv1, sv2, dv0, dv1, dv2, wv0, wv1, wv2,
             g0, g1, g2, s0, s1, s2, i0, i1, i2):
    cid = lax.axis_index("c")
    sid = lax.axis_index("s")
    gwid = cid * NS + sid
    rows = (r0, r1, r2)
    srcb = (sv0, sv1, sv2)
    dstb = (dv0, dv1, dv2)
    wb_ = (wv0, wv1, wv2)
    gsem = (g0, g1, g2)
    ssem = (s0, s1, s2)
    isem = (i0, i1, i2)

    def _zero_row(r, _):
        for j in range(D // LANES):
            r0[r, pl.ds(j * LANES, LANES)] = jnp.zeros((LANES,), jnp.float32)
        return 0
    lax.fori_loop(0, KB, _zero_row, 0)

    def _zinit(i, _):
        blk = sid + i * NS

        @pl.when(blk < NZ)
        def _():
            pltpu.sync_copy(r0, acc.at[pl.ds(blk * ZCH, ZCH)])
        return 0
    lax.fori_loop(0, (NZ + NS - 1) // NS, _zinit, 0)

    @pl.when(sid == 0)
    def _():
        pltpu.sync_copy(r0.at[pl.ds(0, NREM)],
                        acc.at[pl.ds(NZ * ZCH, NREM)])
    plsc.subcore_barrier()

    def _stage_start(blk, b):
        pltpu.async_copy(src_hbm.at[gwid, blk], srcb[b], isem[b])
        pltpu.async_copy(dst_hbm.at[gwid, blk], dstb[b], isem[b])
        pltpu.async_copy(w_hbm.at[gwid, blk], wb_[b], isem[b])

    def _stage_wait(blk, b):
        pltpu.make_async_copy(src_hbm.at[gwid, blk], srcb[b], isem[b]).wait()
        pltpu.make_async_copy(dst_hbm.at[gwid, blk], dstb[b], isem[b]).wait()
        pltpu.make_async_copy(w_hbm.at[gwid, blk], wb_[b], isem[b]).wait()

    def _gather_start(blk, b):
        pltpu.async_copy(h_hbm.at[srcb[b]], rows[b], gsem[b])

    def _gather_wait(blk, b):
        pltpu.make_async_copy(h_hbm.at[srcb[b]], rows[b], gsem[b]).wait()

    def _scatter_start(blk, b):
        pltpu.async_copy(rows[b], acc.at[dstb[b]], ssem[b], add=True)

    def _scatter_wait(blk, b):
        pltpu.make_async_copy(rows[b], acc.at[dstb[b]], ssem[b]).wait()

    _stage_start(0, 0)
    _stage_start(1, 1)
    _stage_wait(0, 0)
    _gather_start(0, 0)
    _stage_wait(1, 1)

    def _trip(q, _):
        for b in range(3):
            blk = 3 * q + b
            bn = (b + 1) % 3
            bp = (b + 2) % 3

            _gather_wait(blk, b)
            if b < 2:
                _gather_start(blk + 1, bn)
            else:
                @pl.when(blk + 1 < NB)
                def _():
                    _gather_start(blk + 1, bn)

            def _scale(c, _):
                w16 = wb_[b][pl.ds(c * LANES, LANES)]
                for lane in range(LANES):
                    wbc = w16.at[jnp.full((LANES,), lane, jnp.int32)].get(
                        mode="promise_in_bounds")
                    e = c * LANES + lane
                    for jj in range(D // LANES):
                        sl = pl.ds(jj * LANES, LANES)
                        rows[b][e, sl] = rows[b][e, sl] * wbc
                return 0
            lax.fori_loop(0, KB // LANES, _scale, 0)

            _scatter_start(blk, b)

            if b == 0:
                @pl.when(blk >= 1)
                def _():
                    _scatter_wait(blk - 1, bp)
            else:
                _scatter_wait(blk - 1, bp)

            if b == 0:
                _stage_start(blk + 2, bp)
            else:
                @pl.when(blk + 2 < NB)
                def _():
                    _stage_start(blk + 2, bp)

            if b == 0:
                _stage_wait(blk + 2, bp)
            else:
                @pl.when(blk + 2 < NB)
                def _():
                    _stage_wait(blk + 2, bp)
        return 0

    lax.fori_loop(0, NB // 3, _trip, 0)
    _scatter_wait(NB - 1, (NB - 1) % 3)
    plsc.subcore_barrier()

    def _wout(i, _):
        blk = sid + i * NS

        @pl.when(blk < NZ)
        def _():
            pltpu.sync_copy(acc.at[pl.ds(blk * ZCH, ZCH)],
                            out_hbm.at[cid, pl.ds(blk * ZCH, ZCH)])
        return 0
    lax.fori_loop(0, (NZ + NS - 1) // NS, _wout, 0)

    @pl.when(sid == 0)
    def _():
        pltpu.sync_copy(acc.at[pl.ds(NZ * ZCH, NREM)],
                        out_hbm.at[cid, pl.ds(NZ * ZCH, NREM)])


def _sc_aggregate(h, src, dst, w):
    mesh = plsc.VectorSubcoreMesh(core_axis_name="c", subcore_axis_name="s")
    f = pl.kernel(
        _sc_body,
        out_type=jax.ShapeDtypeStruct((NC, N, D), jnp.float32),
        mesh=mesh,
        scratch_types=(
            [pltpu.VMEM_SHARED((N, D), jnp.float32)]
            + [pltpu.VMEM((KB, D), jnp.float32) for _ in range(3)]
            + [pltpu.VMEM((KB,), jnp.int32) for _ in range(3)]
            + [pltpu.VMEM((KB,), jnp.int32) for _ in range(3)]
            + [pltpu.VMEM((KB,), jnp.float32) for _ in range(3)]
            + [pltpu.SemaphoreType.DMA for _ in range(9)]
        ),
    )
    return f(h, src, dst, w)


def _comb_body(p_ref, b_ref, o_ref):
    o_ref[...] = p_ref[0] + p_ref[1] + b_ref[...]


def _combine(partials, b):
    m_blk = 1000
    return pl.pallas_call(
        _comb_body,
        grid=(N // m_blk,),
        in_specs=[
            pl.BlockSpec((NC, m_blk, D), lambda i: (0, i, 0)),
            pl.BlockSpec((1, D), lambda i: (0, 0)),
        ],
        out_specs=pl.BlockSpec((m_blk, D), lambda i: (i, 0)),
        out_shape=jax.ShapeDtypeStruct((N, D), jnp.float32),
    )(partials, b.reshape(1, D))


@jax.jit
def kernel(x, edge_index, edge_weight, W, b):
    h = _matmul(x, W)
    pad = EPAD - E
    src = jnp.concatenate([edge_index[0], jnp.zeros((pad,), jnp.int32)])
    dst = jnp.concatenate([edge_index[1], jnp.zeros((pad,), jnp.int32)])
    w = jnp.concatenate([edge_weight, jnp.zeros((pad,), jnp.float32)])
    partials = _sc_aggregate(h, src.reshape(NW, NB, KB),
                             dst.reshape(NW, NB, KB), w.reshape(NW, NB, KB))
    return _combine(partials, b)
